# baseline (speedup 1.0000x reference)
import functools

import jax
import jax.numpy as jnp
from jax import lax
from jax.experimental import pallas as pl
from jax.experimental.pallas import tpu as pltpu
from jax.experimental.pallas import tpu_sc as plsc

BATCH = 16384
D = 64
CH = 128


def kernel(user, item_i, item_j, embed_user, embed_item):
    info = plsc.get_sparse_core_info()
    NC, NS = info.num_cores, info.num_subcores
    NW = NC * NS
    BPW = BATCH // NW
    NCH = BPW // CH

    idx_u = user.reshape(NW, NCH, CH)
    idx_i = item_i.reshape(NW, NCH, CH)
    idx_j = item_j.reshape(NW, NCH, CH)

    mesh = plsc.VectorSubcoreMesh(core_axis_name="c", subcore_axis_name="s")

    @functools.partial(
        pl.kernel,
        out_type=(jax.ShapeDtypeStruct((BATCH,), jnp.float32),
                  jax.ShapeDtypeStruct((BATCH,), jnp.float32)),
        mesh=mesh,
        compiler_params=pltpu.CompilerParams(
            needs_layout_passes=False, use_tc_tiling_on_sc=False),
        scratch_types=[
            pltpu.VMEM((NCH, CH), jnp.int32),
            pltpu.VMEM((NCH, CH), jnp.int32),
            pltpu.VMEM((NCH, CH), jnp.int32),
            pltpu.VMEM((BPW, D), jnp.float32),
            pltpu.VMEM((BPW, D), jnp.float32),
            pltpu.VMEM((BPW, D), jnp.float32),
            pltpu.VMEM((BPW,), jnp.float32),
            pltpu.VMEM((BPW,), jnp.float32),
            pltpu.SemaphoreType.DMA,
        ],
    )
    def bprmf(u_hbm, ii_hbm, ij_hbm, eu_hbm, ei_hbm, oi_hbm, oj_hbm,
              iu_v, ii_v, ij_v, ru_v, ri_v, rj_v, oi_v, oj_v, sem):
        wid = lax.axis_index("s") * NC + lax.axis_index("c")
        pltpu.sync_copy(u_hbm.at[wid], iu_v)
        pltpu.sync_copy(ii_hbm.at[wid], ii_v)
        pltpu.sync_copy(ij_hbm.at[wid], ij_v)

        copies = []
        for c in range(NCH):
            copies.append(pltpu.async_copy(
                eu_hbm.at[iu_v.at[c]], ru_v.at[pl.ds(c * CH, CH)], sem))
            copies.append(pltpu.async_copy(
                ei_hbm.at[ii_v.at[c]], ri_v.at[pl.ds(c * CH, CH)], sem))
            copies.append(pltpu.async_copy(
                ei_hbm.at[ij_v.at[c]], rj_v.at[pl.ds(c * CH, CH)], sem))
        for cp in copies:
            cp.wait()

        iota16 = lax.iota(jnp.int32, 16)

        def body(g, carry):
            rows = g * 16 + iota16
            acc_i = jnp.zeros((16,), jnp.float32)
            acc_j = jnp.zeros((16,), jnp.float32)
            for d in range(D):
                cols = jnp.full((16,), d, jnp.int32)
                uu = plsc.load_gather(ru_v, [rows, cols])
                vi = plsc.load_gather(ri_v, [rows, cols])
                vj = plsc.load_gather(rj_v, [rows, cols])
                acc_i = acc_i + uu * vi
                acc_j = acc_j + uu * vj
            off = pl.multiple_of(g * 16, 16)
            oi_v[pl.ds(off, 16)] = acc_i
            oj_v[pl.ds(off, 16)] = acc_j
            return carry

        lax.fori_loop(0, BPW // 16, body, 0)

        obase = pl.multiple_of(wid * BPW, BPW)
        pltpu.sync_copy(oi_v, oi_hbm.at[pl.ds(obase, BPW)])
        pltpu.sync_copy(oj_v, oj_hbm.at[pl.ds(obase, BPW)])

    return bprmf(idx_u, idx_i, idx_j, embed_user, embed_item)


# speedup vs baseline: 1.5257x; 1.5257x over previous
import functools

import jax
import jax.numpy as jnp
from jax import lax
from jax.experimental import pallas as pl
from jax.experimental.pallas import tpu as pltpu
from jax.experimental.pallas import tpu_sc as plsc

BATCH = 16384
D = 64
CR = 128


def kernel(user, item_i, item_j, embed_user, embed_item):
    info = plsc.get_sparse_core_info()
    NC, NS = info.num_cores, info.num_subcores
    NW = NC * NS
    BPW = BATCH // NW
    NCHK = BPW // CR

    u2 = user.reshape(NW, BPW)
    i2 = item_i.reshape(NW, BPW)
    j2 = item_j.reshape(NW, BPW)

    mesh = plsc.VectorSubcoreMesh(core_axis_name="c", subcore_axis_name="s")

    @functools.partial(
        pl.kernel,
        out_type=(jax.ShapeDtypeStruct((BATCH,), jnp.float32),
                  jax.ShapeDtypeStruct((BATCH,), jnp.float32)),
        mesh=mesh,
        compiler_params=pltpu.CompilerParams(needs_layout_passes=False),
        scratch_types=[
            pltpu.VMEM((BPW,), jnp.int32),
            pltpu.VMEM((BPW,), jnp.int32),
            pltpu.VMEM((BPW,), jnp.int32),
            pltpu.VMEM((CR, D), jnp.float32),
            pltpu.VMEM((CR, D), jnp.float32),
            pltpu.VMEM((CR, D), jnp.float32),
            pltpu.VMEM((BPW,), jnp.float32),
            pltpu.VMEM((BPW,), jnp.float32),
            pltpu.SemaphoreType.DMA,
        ],
    )
    def bprmf(u_hbm, ii_hbm, ij_hbm, eu_hbm, ei_hbm, oi_hbm, oj_hbm,
              ru_v, ri_v, rj_v, gu_v, gi_v, gj_v, oi_v, oj_v, sem):
        wid = lax.axis_index("s") * NC + lax.axis_index("c")
        pltpu.sync_copy(u_hbm.at[wid], ru_v)
        pltpu.sync_copy(ii_hbm.at[wid], ri_v)
        pltpu.sync_copy(ij_hbm.at[wid], rj_v)

        iota16 = lax.iota(jnp.int32, 16)

        def chunk_body(c, carry):
            def fire_body(g, carry2):
                base = pl.multiple_of(c * CR + g * 16, 16)
                uvec = ru_v[pl.ds(base, 16)]
                ivec = ri_v[pl.ds(base, 16)]
                jvec = rj_v[pl.ds(base, 16)]
                for l in range(16):
                    k = g * 16 + l
                    pltpu.async_copy(eu_hbm.at[uvec[l]], gu_v.at[k], sem)
                    pltpu.async_copy(ei_hbm.at[ivec[l]], gi_v.at[k], sem)
                    pltpu.async_copy(ei_hbm.at[jvec[l]], gj_v.at[k], sem)
                return carry2

            lax.fori_loop(0, CR // 16, fire_body, 0)
            pltpu.make_async_copy(eu_hbm.at[pl.ds(0, CR)], gu_v, sem).wait()
            pltpu.make_async_copy(eu_hbm.at[pl.ds(0, CR)], gi_v, sem).wait()
            pltpu.make_async_copy(eu_hbm.at[pl.ds(0, CR)], gj_v, sem).wait()

            def group_body(g, carry2):
                items = g * 16 + iota16
                acc_i = jnp.zeros((16,), jnp.float32)
                acc_j = jnp.zeros((16,), jnp.float32)
                for d in range(D):
                    cols = jnp.full((16,), d, jnp.int32)
                    uu = plsc.load_gather(gu_v, [items, cols])
                    vi = plsc.load_gather(gi_v, [items, cols])
                    vj = plsc.load_gather(gj_v, [items, cols])
                    acc_i = acc_i + uu * vi
                    acc_j = acc_j + uu * vj
                off = pl.multiple_of(c * CR + g * 16, 16)
                oi_v[pl.ds(off, 16)] = acc_i
                oj_v[pl.ds(off, 16)] = acc_j
                return carry2

            lax.fori_loop(0, CR // 16, group_body, 0)
            return carry

        lax.fori_loop(0, NCHK, chunk_body, 0)

        obase = pl.multiple_of(wid * BPW, BPW)
        pltpu.sync_copy(oi_v, oi_hbm.at[pl.ds(obase, BPW)])
        pltpu.sync_copy(oj_v, oj_hbm.at[pl.ds(obase, BPW)])

    return bprmf(u2, i2, j2, embed_user, embed_item)
